# baseline (speedup 1.0000x reference)
import functools

import jax
import jax.numpy as jnp
from jax import lax
from jax.experimental import pallas as pl
from jax.experimental.pallas import tpu as pltpu
from jax.experimental.pallas import tpu_sc as plsc

_B, _L, _N, _C = 8, 12, 2048, 1
_DIM = 24
_ADIM = 80
_TS = 24 * 12
_DOW = 7
_TOT = _B * _L * _N
_OUT_D = 3 * _DIM + _ADIM
_LN = _L * _N
_NC = 2
_NS = 16
_NW = _NC * _NS
_TPW = _TOT // _NW
_T = 512
_NCH = _TPW // _T
_NG = _T // 16
_NSUB = _T // 128


def _sc_body(inp_ref, ctab_ref, wb_ref, adp_ref, out_ref,
             inp_v0, inp_v1, cidx_v0, cidx_v1, femb_v0, femb_v1,
             rows_v0, rows_v1, wb_v, si0, si1, sg0, sg1, so0, so1):
    wid = lax.axis_index("s") * _NC + lax.axis_index("c")
    base0 = wid * _TPW
    pltpu.sync_copy(wb_ref, wb_v)

    lane = lax.iota(jnp.int32, 16)
    lane3 = lane * 3
    wds = [plsc.load_gather(wb_v, [jnp.full((16,), 1 + d, jnp.int32)])
           for d in range(_DIM)]
    bds = [plsc.load_gather(wb_v, [jnp.full((16,), 1 + _DIM + d, jnp.int32)])
           for d in range(_DIM)]

    bufs = ((inp_v0, cidx_v0, femb_v0, rows_v0, si0, sg0, so0),
            (inp_v1, cidx_v1, femb_v1, rows_v1, si1, sg1, so1))

    for p in (0, 1):
        pltpu.async_copy(
            inp_ref.at[pl.ds((base0 + p * _T) * 3, _T * 3)],
            bufs[p][0], bufs[p][4])

    def pair_body(j, carry):
        for p in (0, 1):
            inp_v, cidx_v, femb_v, rows_v, s_in, s_g, s_out = bufs[p]
            i = 2 * j + p
            t0 = base0 + i * _T

            @pl.when(j >= 1)
            def _():
                pltpu.make_async_copy(
                    out_ref.at[pl.ds(0, _T), pl.ds(0, _DIM)],
                    femb_v, s_out).wait()
                pltpu.make_async_copy(
                    out_ref.at[pl.ds(0, _T), pl.ds(24, 48)],
                    rows_v, s_out).wait()
                pltpu.make_async_copy(
                    out_ref.at[pl.ds(0, _T), pl.ds(72, _ADIM)],
                    out_ref.at[pl.ds(0, _T), pl.ds(72, _ADIM)],
                    s_out).wait()

            arow0 = lax.rem(t0, _LN)
            pltpu.async_copy(
                adp_ref.at[pl.ds(arow0, _T)],
                out_ref.at[pl.ds(t0, _T), pl.ds(72, _ADIM)], s_out)

            pltpu.make_async_copy(
                inp_ref.at[pl.ds(0, _T * 3)], inp_v, s_in).wait()

            for g in range(_NG):
                base = g * 48
                tsv = plsc.load_gather(inp_v, [lane3 + (base + 1)])
                dwv = plsc.load_gather(inp_v, [lane3 + (base + 2)])
                comb = tsv.astype(jnp.int32) * _DOW + dwv.astype(jnp.int32)
                cidx_v[g // 8, pl.ds((g % 8) * 16, 16)] = comb

            gcps = [pltpu.async_copy(ctab_ref.at[cidx_v.at[k]],
                                     rows_v.at[pl.ds(k * 128, 128)], s_g)
                    for k in range(_NSUB)]

            for g in range(_NG):
                feat = plsc.load_gather(inp_v, [lane3 + g * 48])
                tok16 = jnp.full((16,), g * 16, jnp.int32) + lane
                for d in range(_DIM):
                    val = feat * wds[d] + bds[d]
                    plsc.store_scatter(
                        femb_v, [tok16, jnp.full((16,), d, jnp.int32)], val)

            nxt = base0 + lax.rem(i + 2, _NCH) * _T
            pltpu.async_copy(inp_ref.at[pl.ds(nxt * 3, _T * 3)], inp_v, s_in)

            pltpu.async_copy(
                femb_v, out_ref.at[pl.ds(t0, _T), pl.ds(0, _DIM)], s_out)
            for cp in gcps:
                cp.wait()
            pltpu.async_copy(
                rows_v, out_ref.at[pl.ds(t0, _T), pl.ds(24, 48)], s_out)
        return carry

    lax.fori_loop(0, _NCH // 2, pair_body, 0)

    for p in (0, 1):
        inp_v, cidx_v, femb_v, rows_v, s_in, s_g, s_out = bufs[p]
        pltpu.make_async_copy(
            out_ref.at[pl.ds(0, _T), pl.ds(0, _DIM)], femb_v, s_out).wait()
        pltpu.make_async_copy(
            out_ref.at[pl.ds(0, _T), pl.ds(24, 48)], rows_v, s_out).wait()
        pltpu.make_async_copy(
            out_ref.at[pl.ds(0, _T), pl.ds(72, _ADIM)],
            out_ref.at[pl.ds(0, _T), pl.ds(72, _ADIM)], s_out).wait()
        pltpu.make_async_copy(
            inp_ref.at[pl.ds(0, _T * 3)], inp_v, s_in).wait()


@jax.jit
def kernel(input, W, b, ts_table, dow_table, adaptive):
    inp_flat = input.reshape(-1)
    wb = jnp.concatenate([jnp.zeros((1,), jnp.float32),
                          W.reshape(-1), b])
    ctab = jnp.concatenate([
        jnp.broadcast_to(ts_table[:, None, :], (_TS, _DOW, _DIM)),
        jnp.broadcast_to(dow_table[None, :, :], (_TS, _DOW, _DIM)),
    ], axis=-1).reshape(_TS * _DOW, 2 * _DIM)
    adp_flat = adaptive.reshape(_LN, _ADIM)

    mesh = plsc.VectorSubcoreMesh(core_axis_name="c", subcore_axis_name="s")
    fn = pl.kernel(
        _sc_body,
        out_type=jax.ShapeDtypeStruct((_TOT, _OUT_D), jnp.float32),
        mesh=mesh,
        compiler_params=pltpu.CompilerParams(use_tc_tiling_on_sc=False,
                                             needs_layout_passes=False),
        scratch_types=[
            pltpu.VMEM((_T * 3,), jnp.float32),
            pltpu.VMEM((_T * 3,), jnp.float32),
            pltpu.VMEM((_NSUB, 128), jnp.int32),
            pltpu.VMEM((_NSUB, 128), jnp.int32),
            pltpu.VMEM((_T, _DIM), jnp.float32),
            pltpu.VMEM((_T, _DIM), jnp.float32),
            pltpu.VMEM((_T, 48), jnp.float32),
            pltpu.VMEM((_T, 48), jnp.float32),
            pltpu.VMEM((2 * _DIM + 1,), jnp.float32),
            pltpu.SemaphoreType.DMA,
            pltpu.SemaphoreType.DMA,
            pltpu.SemaphoreType.DMA,
            pltpu.SemaphoreType.DMA,
            pltpu.SemaphoreType.DMA,
            pltpu.SemaphoreType.DMA,
        ],
    )
    out = fn(inp_flat, ctab, wb, adp_flat)
    return out.reshape(_B, _L, _N, _OUT_D)


# speedup vs baseline: 4.6347x; 4.6347x over previous
import functools

import jax
import jax.numpy as jnp
from jax import lax
from jax.experimental import pallas as pl
from jax.experimental.pallas import tpu as pltpu
from jax.experimental.pallas import tpu_sc as plsc

_B, _L, _N, _C = 8, 12, 2048, 1
_DIM = 24
_ADIM = 80
_TS = 24 * 12
_DOW = 7
_TOT = _B * _L * _N
_OUT_D = 3 * _DIM + _ADIM
_LN = _L * _N
_NC = 2
_NS = 16
_NW = _NC * _NS
_TPW = _TOT // _NW
_T = 256
_NCH = _TPW // _T
_NG = _T // 16
_NSUB = _T // 128


def _sc_body(inp_ref, ctab_ref, wb_ref, adp_ref, out_ref,
             inp_v0, inp_v1, cidx_v0, cidx_v1, femb_v0, femb_v1,
             rows_v0, rows_v1, adp_v0, adp_v1, wb_v,
             si0, si1, sg0, sg1, sa0, sa1, so0, so1):
    wid = lax.axis_index("s") * _NC + lax.axis_index("c")
    base0 = wid * _TPW
    pltpu.sync_copy(wb_ref, wb_v)

    lane = lax.iota(jnp.int32, 16)
    lane3 = lane * 3
    wds = [plsc.load_gather(wb_v, [jnp.full((16,), 1 + d, jnp.int32)])
           for d in range(_DIM)]
    bds = [plsc.load_gather(wb_v, [jnp.full((16,), 1 + _DIM + d, jnp.int32)])
           for d in range(_DIM)]

    bufs = ((inp_v0, cidx_v0, femb_v0, rows_v0, adp_v0, si0, sg0, sa0, so0),
            (inp_v1, cidx_v1, femb_v1, rows_v1, adp_v1, si1, sg1, sa1, so1))

    for p in (0, 1):
        t0p = base0 + p * _T
        pltpu.async_copy(inp_ref.at[pl.ds(t0p * 3, _T * 3)],
                         bufs[p][0], bufs[p][5])

    def pair_body(j, carry):
        for p in (0, 1):
            (inp_v, cidx_v, femb_v, rows_v, adp_v,
             s_in, s_g, s_adp, s_out) = bufs[p]
            i = 2 * j + p
            t0 = base0 + i * _T

            @pl.when(j >= 1)
            def _():
                pltpu.make_async_copy(
                    out_ref.at[pl.ds(0, _T), pl.ds(0, _DIM)],
                    femb_v, s_out).wait()
                pltpu.make_async_copy(
                    out_ref.at[pl.ds(0, _T), pl.ds(24, 48)],
                    rows_v, s_out).wait()
                pltpu.make_async_copy(
                    out_ref.at[pl.ds(0, _T), pl.ds(72, _ADIM)],
                    adp_v, s_out).wait()

            arow0 = lax.rem(t0, _LN)
            pltpu.async_copy(adp_ref.at[pl.ds(arow0, _T)], adp_v, s_adp)

            pltpu.make_async_copy(
                inp_ref.at[pl.ds(0, _T * 3)], inp_v, s_in).wait()

            for g in range(_NG):
                base = g * 48
                tsv = plsc.load_gather(inp_v, [lane3 + (base + 1)])
                dwv = plsc.load_gather(inp_v, [lane3 + (base + 2)])
                comb = tsv.astype(jnp.int32) * _DOW + dwv.astype(jnp.int32)
                cidx_v[g // 8, pl.ds((g % 8) * 16, 16)] = comb

            gcps = [pltpu.async_copy(ctab_ref.at[cidx_v.at[k]],
                                     rows_v.at[pl.ds(k * 128, 128)], s_g)
                    for k in range(_NSUB)]

            for g in range(_NG):
                feat = plsc.load_gather(inp_v, [lane3 + g * 48])
                tok16 = jnp.full((16,), g * 16, jnp.int32) + lane
                for d in range(_DIM):
                    val = feat * wds[d] + bds[d]
                    plsc.store_scatter(
                        femb_v, [tok16, jnp.full((16,), d, jnp.int32)], val)

            nxt_i = lax.rem(i + 2, _NCH)
            nxt = base0 + nxt_i * _T
            pltpu.async_copy(inp_ref.at[pl.ds(nxt * 3, _T * 3)], inp_v, s_in)

            pltpu.async_copy(
                femb_v, out_ref.at[pl.ds(t0, _T), pl.ds(0, _DIM)], s_out)

            pltpu.make_async_copy(
                adp_ref.at[pl.ds(0, _T)], adp_v, s_adp).wait()
            pltpu.async_copy(
                adp_v, out_ref.at[pl.ds(t0, _T), pl.ds(72, _ADIM)], s_out)

            for cp in gcps:
                cp.wait()
            pltpu.async_copy(
                rows_v, out_ref.at[pl.ds(t0, _T), pl.ds(24, 48)], s_out)
        return carry

    lax.fori_loop(0, _NCH // 2, pair_body, 0)

    for p in (0, 1):
        (inp_v, cidx_v, femb_v, rows_v, adp_v,
         s_in, s_g, s_adp, s_out) = bufs[p]
        pltpu.make_async_copy(
            out_ref.at[pl.ds(0, _T), pl.ds(0, _DIM)], femb_v, s_out).wait()
        pltpu.make_async_copy(
            out_ref.at[pl.ds(0, _T), pl.ds(24, 48)], rows_v, s_out).wait()
        pltpu.make_async_copy(
            out_ref.at[pl.ds(0, _T), pl.ds(72, _ADIM)], adp_v, s_out).wait()
        pltpu.make_async_copy(
            inp_ref.at[pl.ds(0, _T * 3)], inp_v, s_in).wait()


@jax.jit
def kernel(input, W, b, ts_table, dow_table, adaptive):
    inp_flat = input.reshape(-1)
    wb = jnp.concatenate([jnp.zeros((1,), jnp.float32),
                          W.reshape(-1), b])
    ctab = jnp.concatenate([
        jnp.broadcast_to(ts_table[:, None, :], (_TS, _DOW, _DIM)),
        jnp.broadcast_to(dow_table[None, :, :], (_TS, _DOW, _DIM)),
    ], axis=-1).reshape(_TS * _DOW, 2 * _DIM)
    adp_flat = adaptive.reshape(_LN, _ADIM)

    mesh = plsc.VectorSubcoreMesh(core_axis_name="c", subcore_axis_name="s")
    fn = pl.kernel(
        _sc_body,
        out_type=jax.ShapeDtypeStruct((_TOT, _OUT_D), jnp.float32),
        mesh=mesh,
        compiler_params=pltpu.CompilerParams(use_tc_tiling_on_sc=False,
                                             needs_layout_passes=False),
        scratch_types=[
            pltpu.VMEM((_T * 3,), jnp.float32),
            pltpu.VMEM((_T * 3,), jnp.float32),
            pltpu.VMEM((_NSUB, 128), jnp.int32),
            pltpu.VMEM((_NSUB, 128), jnp.int32),
            pltpu.VMEM((_T, _DIM), jnp.float32),
            pltpu.VMEM((_T, _DIM), jnp.float32),
            pltpu.VMEM((_T, 48), jnp.float32),
            pltpu.VMEM((_T, 48), jnp.float32),
            pltpu.VMEM((_T, _ADIM), jnp.float32),
            pltpu.VMEM((_T, _ADIM), jnp.float32),
            pltpu.VMEM((2 * _DIM + 1,), jnp.float32),
            pltpu.SemaphoreType.DMA,
            pltpu.SemaphoreType.DMA,
            pltpu.SemaphoreType.DMA,
            pltpu.SemaphoreType.DMA,
            pltpu.SemaphoreType.DMA,
            pltpu.SemaphoreType.DMA,
            pltpu.SemaphoreType.DMA,
            pltpu.SemaphoreType.DMA,
        ],
    )
    out = fn(inp_flat, ctab, wb, adp_flat)
    return out.reshape(_B, _L, _N, _OUT_D)


# speedup vs baseline: 5.4559x; 1.1772x over previous
import functools

import jax
import jax.numpy as jnp
from jax import lax
from jax.experimental import pallas as pl
from jax.experimental.pallas import tpu as pltpu
from jax.experimental.pallas import tpu_sc as plsc

_B, _L, _N, _C = 8, 12, 2048, 1
_DIM = 24
_ADIM = 80
_TS = 24 * 12
_DOW = 7
_TOT = _B * _L * _N
_OUT_D = 3 * _DIM + _ADIM
_LN = _L * _N
_NC = 2
_NS = 16
_NW = _NC * _NS
_TPW = _TOT // _NW
_T = 256
_NCH = _TPW // _T
_NG = _T // 16
_NSUB = _T // 128


def _sc_body(inp_ref, ctab_ref, wb_ref, adp_ref, out_ref,
             inp_v0, inp_v1, cidx_v0, cidx_v1, low_v0, low_v1,
             hi_v, adp_v, wb_v,
             si0, si1, sg0, sg1, sa, so0, so1):
    wid = lax.axis_index("s") * _NC + lax.axis_index("c")
    base0 = wid * _TPW
    pltpu.sync_copy(wb_ref, wb_v)

    lane = lax.iota(jnp.int32, 16)
    lane3 = lane * 3
    wds = [plsc.load_gather(wb_v, [jnp.full((16,), 1 + d, jnp.int32)])
           for d in range(_DIM)]
    bds = [plsc.load_gather(wb_v, [jnp.full((16,), 1 + _DIM + d, jnp.int32)])
           for d in range(_DIM)]

    bufs = ((inp_v0, cidx_v0, low_v0, si0, sg0, so0),
            (inp_v1, cidx_v1, low_v1, si1, sg1, so1))

    def cidx_pass(inp_v, cidx_v):
        for g in range(_NG):
            base = g * 48
            tsv = plsc.load_gather(inp_v, [lane3 + (base + 1)])
            dwv = plsc.load_gather(inp_v, [lane3 + (base + 2)])
            comb = tsv.astype(jnp.int32) * _DOW + dwv.astype(jnp.int32)
            cidx_v[g // 8, pl.ds((g % 8) * 16, 16)] = comb

    def issue_gather(cidx_v, low_v, s_g):
        for k in range(_NSUB):
            pltpu.async_copy(ctab_ref.at[cidx_v.at[k]],
                             low_v.at[pl.ds(k * 128, 128)], s_g)

    pltpu.async_copy(inp_ref.at[pl.ds(base0 * 3, _T * 3)], inp_v0, si0)
    pltpu.async_copy(inp_ref.at[pl.ds((base0 + _T) * 3, _T * 3)],
                     inp_v1, si1)
    pltpu.make_async_copy(inp_ref.at[pl.ds(0, _T * 3)], inp_v0, si0).wait()
    cidx_pass(inp_v0, cidx_v0)
    issue_gather(cidx_v0, low_v0, sg0)

    def pair_body(j, carry):
        for p in (0, 1):
            q = 1 - p
            (inp_v, cidx_v, low_v, s_in, s_g, s_out) = bufs[p]
            (inp_n, cidx_n, low_n, s_in_n, s_g_n, s_out_n) = bufs[q]
            i = 2 * j + p
            t0 = base0 + i * _T

            arow0 = lax.rem(t0, _LN) * _ADIM
            pltpu.async_copy(adp_ref.at[pl.ds(arow0, _T * _ADIM)],
                             adp_v, sa)

            pltpu.make_async_copy(
                ctab_ref.at[pl.ds(0, _T)], low_v, s_g).wait()

            def _drain_prev():
                pltpu.make_async_copy(
                    out_ref.at[pl.ds(0, _T), pl.ds(0, 128)],
                    low_n, s_out_n).wait()
                pltpu.make_async_copy(
                    out_ref.at[pl.ds(0, _T), pl.ds(128, _OUT_D - 128)],
                    hi_v, s_out_n).wait()

            if p == 0:
                pl.when(j >= 1)(_drain_prev)
            else:
                _drain_prev()

            pltpu.make_async_copy(
                inp_ref.at[pl.ds(0, _T * 3)], inp_n, s_in_n).wait()
            cidx_pass(inp_n, cidx_n)
            issue_gather(cidx_n, low_n, s_g_n)

            for g in range(_NG):
                feat = plsc.load_gather(inp_v, [lane3 + g * 48])
                tok16 = jnp.full((16,), g * 16, jnp.int32) + lane
                for d in range(_DIM):
                    val = feat * wds[d] + bds[d]
                    plsc.store_scatter(
                        low_v, [tok16, jnp.full((16,), d, jnp.int32)], val)

            nxt = base0 + lax.rem(i + 2, _NCH) * _T
            pltpu.async_copy(inp_ref.at[pl.ds(nxt * 3, _T * 3)], inp_v, s_in)

            pltpu.make_async_copy(
                adp_ref.at[pl.ds(0, _T * _ADIM)], adp_v, sa).wait()

            def move_rows(r16, carry2):
                for rr in range(16):
                    t = r16 * 16 + rr
                    t80 = t * _ADIM
                    low_v[t, pl.ds(72, 16)] = adp_v[pl.ds(t80, 16)]
                    low_v[t, pl.ds(88, 16)] = adp_v[pl.ds(t80 + 16, 16)]
                    low_v[t, pl.ds(104, 16)] = adp_v[pl.ds(t80 + 32, 16)]
                    low_v[t, pl.ds(112, 16)] = adp_v[pl.ds(t80 + 40, 16)]
                    hi_v[t, pl.ds(0, 16)] = adp_v[pl.ds(t80 + 56, 16)]
                    hi_v[t, pl.ds(8, 16)] = adp_v[pl.ds(t80 + 64, 16)]
                return carry2

            lax.fori_loop(0, _T // 16, move_rows, 0)

            pltpu.async_copy(
                low_v, out_ref.at[pl.ds(t0, _T), pl.ds(0, 128)], s_out)
            pltpu.async_copy(
                hi_v, out_ref.at[pl.ds(t0, _T), pl.ds(128, _OUT_D - 128)],
                s_out)
        return carry

    lax.fori_loop(0, _NCH // 2, pair_body, 0)

    pltpu.make_async_copy(
        out_ref.at[pl.ds(0, _T), pl.ds(0, 128)], low_v1, so1).wait()
    pltpu.make_async_copy(
        out_ref.at[pl.ds(0, _T), pl.ds(128, _OUT_D - 128)],
        hi_v, so1).wait()
    pltpu.make_async_copy(
        inp_ref.at[pl.ds(0, _T * 3)], inp_v1, si1).wait()
    pltpu.make_async_copy(
        ctab_ref.at[pl.ds(0, _T)], low_v0, sg0).wait()


@jax.jit
def kernel(input, W, b, ts_table, dow_table, adaptive):
    inp_flat = input.reshape(-1)
    wb = jnp.concatenate([jnp.zeros((1,), jnp.float32),
                          W.reshape(-1), b,
                          jnp.zeros((7,), jnp.float32)])
    ctab = jnp.concatenate([
        jnp.zeros((_TS, _DOW, _DIM), jnp.float32),
        jnp.broadcast_to(ts_table[:, None, :], (_TS, _DOW, _DIM)),
        jnp.broadcast_to(dow_table[None, :, :], (_TS, _DOW, _DIM)),
        jnp.zeros((_TS, _DOW, 56), jnp.float32),
    ], axis=-1).reshape(_TS * _DOW, 128)
    adp1 = adaptive.reshape(-1)

    mesh = plsc.VectorSubcoreMesh(core_axis_name="c", subcore_axis_name="s")
    fn = pl.kernel(
        _sc_body,
        out_type=jax.ShapeDtypeStruct((_TOT, _OUT_D), jnp.float32),
        mesh=mesh,
        compiler_params=pltpu.CompilerParams(use_tc_tiling_on_sc=True,
                                             needs_layout_passes=False),
        scratch_types=[
            pltpu.VMEM((_T * 3,), jnp.float32),
            pltpu.VMEM((_T * 3,), jnp.float32),
            pltpu.VMEM((_NSUB, 128), jnp.int32),
            pltpu.VMEM((_NSUB, 128), jnp.int32),
            pltpu.VMEM((_T, 128), jnp.float32),
            pltpu.VMEM((_T, 128), jnp.float32),
            pltpu.VMEM((_T, _OUT_D - 128), jnp.float32),
            pltpu.VMEM((_T * _ADIM,), jnp.float32),
            pltpu.VMEM((56,), jnp.float32),
            pltpu.SemaphoreType.DMA,
            pltpu.SemaphoreType.DMA,
            pltpu.SemaphoreType.DMA,
            pltpu.SemaphoreType.DMA,
            pltpu.SemaphoreType.DMA,
            pltpu.SemaphoreType.DMA,
            pltpu.SemaphoreType.DMA,
        ],
    )
    out = fn(inp_flat, ctab, wb, adp1)
    return out.reshape(_B, _L, _N, _OUT_D)


# speedup vs baseline: 8.8882x; 1.6291x over previous
import functools

import jax
import jax.numpy as jnp
from jax import lax
from jax.experimental import pallas as pl
from jax.experimental.pallas import tpu as pltpu
from jax.experimental.pallas import tpu_sc as plsc

_B, _L, _N, _C = 8, 12, 2048, 1
_DIM = 24
_ADIM = 80
_TS = 24 * 12
_DOW = 7
_TOT = _B * _L * _N
_OUT_D = 3 * _DIM + _ADIM
_NP = _B * _L
_NC = 2
_NS = 16
_NW = _NC * _NS
_TPW = _TOT // _NW
_BLK = 128
_NBLK = _TPW // _BLK


def _sc_body(inp_ref, ts_ref, dow_ref, wb_ref, adp_ref, out_ref,
             inp_v0, inp_v1, asm_v0, asm_v1, ts_v, dow_v, wb_v,
             si0, si1, sa, so0, so1):
    wid = lax.axis_index("s") * _NC + lax.axis_index("c")
    base0 = wid * _TPW
    pltpu.sync_copy(wb_ref, wb_v)
    pltpu.sync_copy(ts_ref, ts_v)
    pltpu.sync_copy(dow_ref, dow_v)

    lane = lax.iota(jnp.int32, 16)
    lane3 = lane * 3
    wds = [plsc.load_gather(wb_v, [jnp.full((16,), 1 + d, jnp.int32)])
           for d in range(_DIM)]
    bds = [plsc.load_gather(wb_v, [jnp.full((16,), 1 + _DIM + d, jnp.int32)])
           for d in range(_DIM)]

    bufs = ((inp_v0, asm_v0, si0, so0),
            (inp_v1, asm_v1, si1, so1))

    def pair_body(j, carry):
        for p in (0, 1):
            q = 1 - p
            inp_v, asm_v, s_in, s_out = bufs[p]
            inp_n, asm_n, s_in_n, s_out_n = bufs[q]
            k = 2 * j + p
            t0 = base0 + k * _BLK
            plane = t0 // _N
            n0 = lax.rem(t0, _N)

            def _drain_prev():
                pltpu.make_async_copy(
                    out_ref.at[0, :, pl.ds(0, _BLK)], asm_n, s_out_n).wait()

            if p == 0:
                pl.when(j >= 1)(_drain_prev)
            else:
                _drain_prev()

            lrow = lax.rem(plane, _L) * _ADIM
            pltpu.async_copy(
                adp_ref.at[pl.ds(lrow, _ADIM), pl.ds(n0, _BLK)],
                asm_v.at[pl.ds(72, _ADIM), :], sa)

            pltpu.make_async_copy(
                inp_ref.at[pl.ds(0, _BLK * 3)], inp_v, s_in).wait()

            for g in range(_BLK // 16):
                base = g * 48
                feat = plsc.load_gather(inp_v, [lane3 + base])
                tsv = plsc.load_gather(inp_v, [lane3 + (base + 1)])
                dwv = plsc.load_gather(inp_v, [lane3 + (base + 2)])
                ts24 = tsv.astype(jnp.int32) * _DIM
                dw24 = dwv.astype(jnp.int32) * _DIM
                sl = pl.ds(g * 16, 16)
                for d in range(_DIM):
                    asm_v[d, sl] = feat * wds[d] + bds[d]
                for d in range(_DIM):
                    asm_v[24 + d, sl] = plsc.load_gather(ts_v, [ts24 + d])
                for d in range(_DIM):
                    asm_v[48 + d, sl] = plsc.load_gather(dow_v, [dw24 + d])

            nxt = base0 + lax.rem(k + 2, _NBLK) * _BLK
            pltpu.async_copy(inp_ref.at[pl.ds(nxt * 3, _BLK * 3)],
                             inp_v, s_in)

            pltpu.make_async_copy(
                adp_ref.at[pl.ds(0, _ADIM), pl.ds(0, _BLK)],
                asm_v.at[pl.ds(72, _ADIM), :], sa).wait()
            pltpu.async_copy(
                asm_v, out_ref.at[plane, :, pl.ds(n0, _BLK)], s_out)
        return carry

    pltpu.async_copy(inp_ref.at[pl.ds(base0 * 3, _BLK * 3)], inp_v0, si0)
    pltpu.async_copy(inp_ref.at[pl.ds((base0 + _BLK) * 3, _BLK * 3)],
                     inp_v1, si1)

    lax.fori_loop(0, _NBLK // 2, pair_body, 0)

    pltpu.make_async_copy(
        out_ref.at[0, :, pl.ds(0, _BLK)], asm_v1, so1).wait()
    pltpu.make_async_copy(
        inp_ref.at[pl.ds(0, _BLK * 3)], inp_v0, si0).wait()
    pltpu.make_async_copy(
        inp_ref.at[pl.ds(0, _BLK * 3)], inp_v1, si1).wait()


@jax.jit
def kernel(input, W, b, ts_table, dow_table, adaptive):
    inp_flat = input.reshape(-1)
    wb = jnp.concatenate([jnp.zeros((1,), jnp.float32),
                          W.reshape(-1), b,
                          jnp.zeros((7,), jnp.float32)])
    ts1 = ts_table.reshape(-1)
    dow1 = dow_table.reshape(-1)
    adp_t = adaptive.transpose(0, 2, 1).reshape(_L * _ADIM, _N)

    mesh = plsc.VectorSubcoreMesh(core_axis_name="c", subcore_axis_name="s")
    fn = pl.kernel(
        _sc_body,
        out_type=jax.ShapeDtypeStruct((_NP, _OUT_D, _N), jnp.float32),
        mesh=mesh,
        compiler_params=pltpu.CompilerParams(use_tc_tiling_on_sc=True,
                                             needs_layout_passes=False),
        scratch_types=[
            pltpu.VMEM((_BLK * 3,), jnp.float32),
            pltpu.VMEM((_BLK * 3,), jnp.float32),
            pltpu.VMEM((_OUT_D, _BLK), jnp.float32),
            pltpu.VMEM((_OUT_D, _BLK), jnp.float32),
            pltpu.VMEM((_TS * _DIM,), jnp.float32),
            pltpu.VMEM((_DOW * _DIM,), jnp.float32),
            pltpu.VMEM((56,), jnp.float32),
            pltpu.SemaphoreType.DMA,
            pltpu.SemaphoreType.DMA,
            pltpu.SemaphoreType.DMA,
            pltpu.SemaphoreType.DMA,
            pltpu.SemaphoreType.DMA,
        ],
    )
    out = fn(inp_flat, ts1, dow1, wb, adp_t)
    out = out.reshape(_B, _L, _OUT_D, _N)
    return out.transpose(0, 1, 3, 2)


# speedup vs baseline: 9.6371x; 1.0843x over previous
import functools

import jax
import jax.numpy as jnp
from jax import lax
from jax.experimental import pallas as pl
from jax.experimental.pallas import tpu as pltpu
from jax.experimental.pallas import tpu_sc as plsc

_B, _L, _N, _C = 8, 12, 2048, 1
_DIM = 24
_ADIM = 80
_TS = 24 * 12
_DOW = 7
_TOT = _B * _L * _N
_OUT_D = 3 * _DIM + _ADIM
_NP = _B * _L
_NC = 2
_NS = 16
_NW = _NC * _NS
_TPW = _TOT // _NW
_BLK = 128
_NBLK = _TPW // _BLK


def _sc_body(inp_ref, ts_ref, dow_ref, wb_ref, adp_ref, out_ref,
             inp_v0, inp_v1, asm_v0, asm_v1, ts_v, dow_v, wb_v,
             si0, si1, sa, so0, so1):
    wid = lax.axis_index("s") * _NC + lax.axis_index("c")
    base0 = wid * _TPW
    pltpu.sync_copy(wb_ref, wb_v)
    pltpu.sync_copy(ts_ref, ts_v)
    pltpu.sync_copy(dow_ref, dow_v)

    lane = lax.iota(jnp.int32, 16)
    lane3 = lane * 3
    wds = [plsc.load_gather(wb_v, [jnp.full((16,), 1 + d, jnp.int32)])
           for d in range(_DIM)]
    bds = [plsc.load_gather(wb_v, [jnp.full((16,), 1 + _DIM + d, jnp.int32)])
           for d in range(_DIM)]

    bufs = ((inp_v0, asm_v0, si0, so0),
            (inp_v1, asm_v1, si1, so1))

    def pair_body(j, carry):
        for p in (0, 1):
            q = 1 - p
            inp_v, asm_v, s_in, s_out = bufs[p]
            inp_n, asm_n, s_in_n, s_out_n = bufs[q]
            k = 2 * j + p
            t0 = base0 + k * _BLK
            plane = t0 // _N
            n0 = lax.rem(t0, _N)

            def _drain_prev():
                pltpu.make_async_copy(
                    out_ref.at[0, :, pl.ds(0, _BLK)], asm_v, s_out).wait()

            pl.when(j >= 1)(_drain_prev)

            lrow = lax.rem(plane, _L) * _ADIM
            pltpu.async_copy(
                adp_ref.at[pl.ds(lrow, _ADIM), pl.ds(n0, _BLK)],
                asm_v.at[pl.ds(72, _ADIM), :], sa)

            pltpu.make_async_copy(
                inp_ref.at[pl.ds(0, _BLK * 3)], inp_v, s_in).wait()

            for g in range(_BLK // 16):
                base = g * 48
                feat = plsc.load_gather(inp_v, [lane3 + base])
                tsv = plsc.load_gather(inp_v, [lane3 + (base + 1)])
                dwv = plsc.load_gather(inp_v, [lane3 + (base + 2)])
                ts24 = tsv.astype(jnp.int32) * _DIM
                dw24 = dwv.astype(jnp.int32) * _DIM
                sl = pl.ds(g * 16, 16)
                for d in range(_DIM):
                    asm_v[d, sl] = feat * wds[d] + bds[d]
                for d in range(_DIM):
                    asm_v[24 + d, sl] = plsc.load_gather(ts_v, [ts24 + d])
                for d in range(_DIM):
                    asm_v[48 + d, sl] = plsc.load_gather(dow_v, [dw24 + d])

            nxt = base0 + lax.rem(k + 2, _NBLK) * _BLK
            pltpu.async_copy(inp_ref.at[pl.ds(nxt * 3, _BLK * 3)],
                             inp_v, s_in)

            pltpu.make_async_copy(
                adp_ref.at[pl.ds(0, _ADIM), pl.ds(0, _BLK)],
                asm_v.at[pl.ds(72, _ADIM), :], sa).wait()
            pltpu.async_copy(
                asm_v, out_ref.at[plane, :, pl.ds(n0, _BLK)], s_out)
        return carry

    pltpu.async_copy(inp_ref.at[pl.ds(base0 * 3, _BLK * 3)], inp_v0, si0)
    pltpu.async_copy(inp_ref.at[pl.ds((base0 + _BLK) * 3, _BLK * 3)],
                     inp_v1, si1)

    lax.fori_loop(0, _NBLK // 2, pair_body, 0)

    pltpu.make_async_copy(
        out_ref.at[0, :, pl.ds(0, _BLK)], asm_v0, so0).wait()
    pltpu.make_async_copy(
        out_ref.at[0, :, pl.ds(0, _BLK)], asm_v1, so1).wait()
    pltpu.make_async_copy(
        inp_ref.at[pl.ds(0, _BLK * 3)], inp_v0, si0).wait()
    pltpu.make_async_copy(
        inp_ref.at[pl.ds(0, _BLK * 3)], inp_v1, si1).wait()


@jax.jit
def kernel(input, W, b, ts_table, dow_table, adaptive):
    inp_flat = input.reshape(-1)
    wb = jnp.concatenate([jnp.zeros((1,), jnp.float32),
                          W.reshape(-1), b,
                          jnp.zeros((7,), jnp.float32)])
    ts1 = ts_table.reshape(-1)
    dow1 = dow_table.reshape(-1)
    adp_t = adaptive.transpose(0, 2, 1).reshape(_L * _ADIM, _N)

    mesh = plsc.VectorSubcoreMesh(core_axis_name="c", subcore_axis_name="s")
    fn = pl.kernel(
        _sc_body,
        out_type=jax.ShapeDtypeStruct((_NP, _OUT_D, _N), jnp.float32),
        mesh=mesh,
        compiler_params=pltpu.CompilerParams(use_tc_tiling_on_sc=True,
                                             needs_layout_passes=False),
        scratch_types=[
            pltpu.VMEM((_BLK * 3,), jnp.float32),
            pltpu.VMEM((_BLK * 3,), jnp.float32),
            pltpu.VMEM((_OUT_D, _BLK), jnp.float32),
            pltpu.VMEM((_OUT_D, _BLK), jnp.float32),
            pltpu.VMEM((_TS * _DIM,), jnp.float32),
            pltpu.VMEM((_DOW * _DIM,), jnp.float32),
            pltpu.VMEM((56,), jnp.float32),
            pltpu.SemaphoreType.DMA,
            pltpu.SemaphoreType.DMA,
            pltpu.SemaphoreType.DMA,
            pltpu.SemaphoreType.DMA,
            pltpu.SemaphoreType.DMA,
        ],
    )
    out = fn(inp_flat, ts1, dow1, wb, adp_t)
    out = out.reshape(_B, _L, _OUT_D, _N)
    return out.transpose(0, 1, 3, 2)
